# baseline (speedup 1.0000x reference)
import functools

import jax
import jax.numpy as jnp
from jax import lax
from jax.experimental import pallas as pl
from jax.experimental.pallas import tpu as pltpu
from jax.experimental.pallas import tpu_sc as plsc

B, L, D = 1, 2048, 1024
E, K, DFF = 8, 2, 4096
N = B * L
SEG = N
CHUNK = 256
FBLK = 256
NF = DFF // FBLK
CSH = 512
NC, NS = 2, 16
NW = NC * NS
TOK_W = N // NW
WREP = 128
CSUB = 32


def _gate_body(x_ref, wg_ref, aux_ref, cnt_ref, pos0_ref, pos1_ref,
               w0_ref, w1_ref):
    x = x_ref[...]
    logits = lax.dot_general(
        x.astype(jnp.bfloat16), wg_ref[...].astype(jnp.bfloat16),
        (((1,), (0,)), ((), ())), preferred_element_type=jnp.float32)
    m = jnp.max(logits, axis=1, keepdims=True)
    ex = jnp.exp(logits - m)
    probs = ex / jnp.sum(ex, axis=1, keepdims=True)

    idxs = lax.broadcasted_iota(jnp.int32, (N, E), 1)
    m1 = jnp.max(probs, axis=1, keepdims=True)
    i1 = jnp.min(jnp.where(probs == m1, idxs, E), axis=1, keepdims=True)
    oh1 = (idxs == i1).astype(jnp.float32)
    probs2 = jnp.where(idxs == i1, -jnp.inf, probs)
    m2 = jnp.max(probs2, axis=1, keepdims=True)
    i2 = jnp.min(jnp.where(probs2 == m2, idxs, E), axis=1, keepdims=True)
    oh2 = (idxs == i2).astype(jnp.float32)

    s = m1 + m2 + 1e-9
    w0_ref[...] = jnp.broadcast_to(m1 / s, (N, WREP))
    w1_ref[...] = jnp.broadcast_to(m2 / s, (N, WREP))

    c = oh1 + oh2
    counts = jnp.sum(c, axis=0, keepdims=True)
    mean_prob = jnp.sum(probs, axis=0, keepdims=True) / N
    aux_ref[...] = jnp.reshape(
        jnp.sum((counts / N) * mean_prob) * E, (1, 1))
    cnt_ref[...] = counts.astype(jnp.int32)

    r_i = lax.broadcasted_iota(jnp.int32, (CSH, CSH), 0)
    c_i = lax.broadcasted_iota(jnp.int32, (CSH, CSH), 1)
    tri = (c_i < r_i).astype(jnp.bfloat16)
    carry = jnp.zeros((1, E), jnp.float32)
    excl_parts = []
    for k in range(N // CSH):
        cc = c[k * CSH:(k + 1) * CSH, :]
        excl_parts.append(lax.dot_general(
            tri, cc.astype(jnp.bfloat16), (((1,), (0,)), ((), ())),
            preferred_element_type=jnp.float32) + carry)
        carry = carry + jnp.sum(cc, axis=0, keepdims=True)
    excl = jnp.concatenate(excl_parts, axis=0)

    posf = excl + SEG * idxs.astype(jnp.float32)
    pos0_ref[...] = jnp.sum(oh1 * posf, axis=1, keepdims=True).astype(jnp.int32)
    pos1_ref[...] = jnp.sum(oh2 * posf, axis=1, keepdims=True).astype(jnp.int32)


def _gate_call(x2, Wg):
    return pl.pallas_call(
        _gate_body,
        in_specs=[pl.BlockSpec((N, D), lambda: (0, 0)),
                  pl.BlockSpec((D, E), lambda: (0, 0))],
        out_specs=[pl.BlockSpec((1, 1), lambda: (0, 0)),
                   pl.BlockSpec((1, E), lambda: (0, 0)),
                   pl.BlockSpec((N, 1), lambda: (0, 0)),
                   pl.BlockSpec((N, 1), lambda: (0, 0)),
                   pl.BlockSpec((N, WREP), lambda: (0, 0)),
                   pl.BlockSpec((N, WREP), lambda: (0, 0))],
        out_shape=[jax.ShapeDtypeStruct((1, 1), jnp.float32),
                   jax.ShapeDtypeStruct((1, E), jnp.int32),
                   jax.ShapeDtypeStruct((N, 1), jnp.int32),
                   jax.ShapeDtypeStruct((N, 1), jnp.int32),
                   jax.ShapeDtypeStruct((N, WREP), jnp.float32),
                   jax.ShapeDtypeStruct((N, WREP), jnp.float32)],
    )(x2, Wg)


@functools.lru_cache(maxsize=1)
def _make_dispatch():
    mesh = plsc.VectorSubcoreMesh(core_axis_name="c", subcore_axis_name="s")

    @functools.partial(
        pl.kernel, mesh=mesh,
        out_type=[jax.ShapeDtypeStruct((E * SEG, D), jnp.float32),
                  jax.ShapeDtypeStruct((E * SEG, WREP), jnp.float32)],
        scratch_types=[pltpu.VMEM((TOK_W, D), jnp.float32),
                       pltpu.VMEM((TOK_W, WREP), jnp.float32),
                       pltpu.VMEM((TOK_W, WREP), jnp.float32),
                       pltpu.VMEM((TOK_W,), jnp.int32),
                       pltpu.VMEM((TOK_W,), jnp.int32),
                       pltpu.SemaphoreType.DMA,
                       pltpu.SemaphoreType.DMA])
    def _dispatch(x_hbm, p0_hbm, p1_hbm, w0_hbm, w1_hbm, xs_hbm, ws_hbm,
                  xbuf, wb0, wb1, i0, i1, sem0, sem1):
        wid = lax.axis_index("s") * NC + lax.axis_index("c")
        base = wid * TOK_W
        pltpu.sync_copy(p0_hbm.at[pl.ds(base, TOK_W)], i0)
        pltpu.sync_copy(p1_hbm.at[pl.ds(base, TOK_W)], i1)
        pltpu.sync_copy(x_hbm.at[pl.ds(base, TOK_W)], xbuf)
        cp0 = pltpu.async_copy(xbuf, xs_hbm.at[i0], sem0)
        cp1 = pltpu.async_copy(xbuf, xs_hbm.at[i1], sem1)
        pltpu.sync_copy(w0_hbm.at[pl.ds(base, TOK_W)], wb0)
        pltpu.sync_copy(w1_hbm.at[pl.ds(base, TOK_W)], wb1)
        cp0.wait()
        cp1.wait()
        cp0 = pltpu.async_copy(wb0, ws_hbm.at[i0], sem0)
        cp1 = pltpu.async_copy(wb1, ws_hbm.at[i1], sem1)
        cp0.wait()
        cp1.wait()

    return _dispatch


def _ffn_body(cnt_ref, xs_ref, ws_ref, w1_ref, b1_ref, w2_ref, b2_ref,
              ys_ref, w1b_ref, w2b_ref, hb_ref):
    e = pl.program_id(0)
    f = pl.program_id(1)
    fo = pl.multiple_of(f * FBLK, FBLK)
    w1b_ref[:, pl.ds(fo, FBLK)] = w1_ref[0].astype(jnp.bfloat16)
    w2b_ref[pl.ds(fo, FBLK), :] = w2_ref[0].astype(jnp.bfloat16)

    @pl.when(f == NF - 1)
    def _():
        cnt = cnt_ref[e]
        trips = lax.div(cnt + (CHUNK - 1), CHUNK)

        def body(i, carry):
            rs = pl.ds(i * CHUNK, CHUNK)
            xb = xs_ref[rs, :].astype(jnp.bfloat16)
            for ff in range(NF):
                hcol = lax.dot_general(
                    xb, w1b_ref[:, ff * FBLK:(ff + 1) * FBLK],
                    (((1,), (0,)), ((), ())),
                    preferred_element_type=jnp.float32)
                hcol = hcol + b1_ref[0][:, ff * FBLK:(ff + 1) * FBLK]
                hcol = 0.5 * hcol * (1.0 + lax.erf(hcol * 0.7071067811865476))
                hb_ref[:, ff * FBLK:(ff + 1) * FBLK] = hcol.astype(jnp.bfloat16)
            y = lax.dot_general(hb_ref[...], w2b_ref[...],
                                (((1,), (0,)), ((), ())),
                                preferred_element_type=jnp.float32)
            ys_ref[rs, :] = ws_ref[rs, 0:1] * (y + b2_ref[0])
            return carry

        lax.fori_loop(0, trips, body, 0)


def _ffn_call(counts, xs, ws, W1, b1r, W2, b2r):
    grid_spec = pltpu.PrefetchScalarGridSpec(
        num_scalar_prefetch=1,
        grid=(E, NF),
        in_specs=[
            pl.BlockSpec((SEG, D), lambda e, f, cnt: (e, 0)),
            pl.BlockSpec((SEG, WREP), lambda e, f, cnt: (e, 0)),
            pl.BlockSpec((1, D, FBLK), lambda e, f, cnt: (e, 0, f)),
            pl.BlockSpec((1, 1, DFF), lambda e, f, cnt: (e, 0, 0)),
            pl.BlockSpec((1, FBLK, D), lambda e, f, cnt: (e, f, 0)),
            pl.BlockSpec((1, 1, D), lambda e, f, cnt: (e, 0, 0)),
        ],
        out_specs=pl.BlockSpec((SEG, D), lambda e, f, cnt: (e, 0)),
        scratch_shapes=[pltpu.VMEM((D, DFF), jnp.bfloat16),
                        pltpu.VMEM((DFF, D), jnp.bfloat16),
                        pltpu.VMEM((CHUNK, DFF), jnp.bfloat16)],
    )
    return pl.pallas_call(
        _ffn_body,
        grid_spec=grid_spec,
        out_shape=jax.ShapeDtypeStruct((E * SEG, D), jnp.float32),
    )(counts, xs, ws, W1, b1r, W2, b2r)


@functools.lru_cache(maxsize=1)
def _make_combine():
    mesh = plsc.VectorSubcoreMesh(core_axis_name="c", subcore_axis_name="s")

    @functools.partial(
        pl.kernel, mesh=mesh,
        out_type=jax.ShapeDtypeStruct((N, D), jnp.float32),
        scratch_types=[pltpu.VMEM((CSUB, D), jnp.float32),
                       pltpu.VMEM((CSUB, D), jnp.float32),
                       pltpu.VMEM((CSUB,), jnp.int32),
                       pltpu.VMEM((CSUB,), jnp.int32),
                       pltpu.SemaphoreType.DMA,
                       pltpu.SemaphoreType.DMA])
    def _combine(ys_hbm, p0_hbm, p1_hbm, out_hbm, b0, b1, i0, i1, sem0, sem1):
        wid = lax.axis_index("s") * NC + lax.axis_index("c")
        base = wid * TOK_W
        for sub in range(TOK_W // CSUB):
            sb = base + sub * CSUB
            pltpu.sync_copy(p0_hbm.at[pl.ds(sb, CSUB)], i0)
            pltpu.sync_copy(p1_hbm.at[pl.ds(sb, CSUB)], i1)
            cp0 = pltpu.async_copy(ys_hbm.at[i0], b0, sem0)
            cp1 = pltpu.async_copy(ys_hbm.at[i1], b1, sem1)
            cp0.wait()
            cp1.wait()

            @pl.loop(0, CSUB)
            def _(r):
                @pl.loop(0, D, step=16)
                def _(cc):
                    b0[r, pl.ds(cc, 16)] = (b0[r, pl.ds(cc, 16)]
                                            + b1[r, pl.ds(cc, 16)])

            pltpu.sync_copy(b0, out_hbm.at[pl.ds(sb, CSUB)])

    return _combine


@jax.jit
def kernel(x, Wg, W1, b1, W2, b2):
    x2 = x.reshape(N, D)
    aux, cnt2, pos0c, pos1c, w0r, w1r = _gate_call(x2, Wg)
    counts = cnt2.reshape(E)
    pos0 = pos0c.reshape(N)
    pos1 = pos1c.reshape(N)
    xs, ws = _make_dispatch()(x2, pos0, pos1, w0r, w1r)
    ys = _ffn_call(counts, xs, ws, W1, b1.reshape(E, 1, DFF), W2,
                   b2.reshape(E, 1, D))
    out = _make_combine()(ys, pos0, pos1)
    return out.reshape(B, L, D), aux[0, 0]


# speedup vs baseline: 1.2772x; 1.2772x over previous
import functools

import jax
import jax.numpy as jnp
from jax import lax
from jax.experimental import pallas as pl
from jax.experimental.pallas import tpu as pltpu
from jax.experimental.pallas import tpu_sc as plsc

B, L, D = 1, 2048, 1024
E, K, DFF = 8, 2, 4096
N = B * L
SEG = N
CHUNK = 256
FBLK = 1024
NF = DFF // FBLK
CSH = 512
NC, NS = 2, 16
NW = NC * NS
TOK_W = N // NW
WREP = 128
CSUB = 32


def _gate_body(x_ref, wg_ref, aux_ref, cnt_ref, pos0_ref, pos1_ref,
               w0_ref, w1_ref):
    x = x_ref[...]
    logits = lax.dot_general(
        x.astype(jnp.bfloat16), wg_ref[...].astype(jnp.bfloat16),
        (((1,), (0,)), ((), ())), preferred_element_type=jnp.float32)
    m = jnp.max(logits, axis=1, keepdims=True)
    ex = jnp.exp(logits - m)
    probs = ex / jnp.sum(ex, axis=1, keepdims=True)

    idxs = lax.broadcasted_iota(jnp.int32, (N, E), 1)
    m1 = jnp.max(probs, axis=1, keepdims=True)
    i1 = jnp.min(jnp.where(probs == m1, idxs, E), axis=1, keepdims=True)
    oh1 = (idxs == i1).astype(jnp.float32)
    probs2 = jnp.where(idxs == i1, -jnp.inf, probs)
    m2 = jnp.max(probs2, axis=1, keepdims=True)
    i2 = jnp.min(jnp.where(probs2 == m2, idxs, E), axis=1, keepdims=True)
    oh2 = (idxs == i2).astype(jnp.float32)

    s = m1 + m2 + 1e-9
    w0_ref[...] = jnp.broadcast_to(m1 / s, (N, WREP))
    w1_ref[...] = jnp.broadcast_to(m2 / s, (N, WREP))

    c = oh1 + oh2
    counts = jnp.sum(c, axis=0, keepdims=True)
    mean_prob = jnp.sum(probs, axis=0, keepdims=True) / N
    aux_ref[...] = jnp.reshape(
        jnp.sum((counts / N) * mean_prob) * E, (1, 1))
    cnt_ref[...] = counts.astype(jnp.int32)

    r_i = lax.broadcasted_iota(jnp.int32, (CSH, CSH), 0)
    c_i = lax.broadcasted_iota(jnp.int32, (CSH, CSH), 1)
    tri = (c_i < r_i).astype(jnp.bfloat16)
    carry = jnp.zeros((1, E), jnp.float32)
    excl_parts = []
    for k in range(N // CSH):
        cc = c[k * CSH:(k + 1) * CSH, :]
        excl_parts.append(lax.dot_general(
            tri, cc.astype(jnp.bfloat16), (((1,), (0,)), ((), ())),
            preferred_element_type=jnp.float32) + carry)
        carry = carry + jnp.sum(cc, axis=0, keepdims=True)
    excl = jnp.concatenate(excl_parts, axis=0)

    posf = excl + SEG * idxs.astype(jnp.float32)
    pos0_ref[...] = jnp.sum(oh1 * posf, axis=1, keepdims=True).astype(jnp.int32)
    pos1_ref[...] = jnp.sum(oh2 * posf, axis=1, keepdims=True).astype(jnp.int32)


def _gate_call(x2, Wg):
    return pl.pallas_call(
        _gate_body,
        in_specs=[pl.BlockSpec((N, D), lambda: (0, 0)),
                  pl.BlockSpec((D, E), lambda: (0, 0))],
        out_specs=[pl.BlockSpec((1, 1), lambda: (0, 0)),
                   pl.BlockSpec((1, E), lambda: (0, 0)),
                   pl.BlockSpec((N, 1), lambda: (0, 0)),
                   pl.BlockSpec((N, 1), lambda: (0, 0)),
                   pl.BlockSpec((N, WREP), lambda: (0, 0)),
                   pl.BlockSpec((N, WREP), lambda: (0, 0))],
        out_shape=[jax.ShapeDtypeStruct((1, 1), jnp.float32),
                   jax.ShapeDtypeStruct((1, E), jnp.int32),
                   jax.ShapeDtypeStruct((N, 1), jnp.int32),
                   jax.ShapeDtypeStruct((N, 1), jnp.int32),
                   jax.ShapeDtypeStruct((N, WREP), jnp.float32),
                   jax.ShapeDtypeStruct((N, WREP), jnp.float32)],
    )(x2, Wg)


@functools.lru_cache(maxsize=1)
def _make_dispatch():
    mesh = plsc.VectorSubcoreMesh(core_axis_name="c", subcore_axis_name="s")

    @functools.partial(
        pl.kernel, mesh=mesh,
        out_type=[jax.ShapeDtypeStruct((E * SEG, D), jnp.float32),
                  jax.ShapeDtypeStruct((E * SEG, WREP), jnp.float32)],
        scratch_types=[pltpu.VMEM((TOK_W, D), jnp.float32),
                       pltpu.VMEM((TOK_W, WREP), jnp.float32),
                       pltpu.VMEM((TOK_W, WREP), jnp.float32),
                       pltpu.VMEM((TOK_W,), jnp.int32),
                       pltpu.VMEM((TOK_W,), jnp.int32),
                       pltpu.SemaphoreType.DMA,
                       pltpu.SemaphoreType.DMA])
    def _dispatch(x_hbm, p0_hbm, p1_hbm, w0_hbm, w1_hbm, xs_hbm, ws_hbm,
                  xbuf, wb0, wb1, i0, i1, sem0, sem1):
        wid = lax.axis_index("s") * NC + lax.axis_index("c")
        base = wid * TOK_W
        pltpu.sync_copy(p0_hbm.at[pl.ds(base, TOK_W)], i0)
        pltpu.sync_copy(p1_hbm.at[pl.ds(base, TOK_W)], i1)
        pltpu.sync_copy(x_hbm.at[pl.ds(base, TOK_W)], xbuf)
        cp0 = pltpu.async_copy(xbuf, xs_hbm.at[i0], sem0)
        cp1 = pltpu.async_copy(xbuf, xs_hbm.at[i1], sem1)
        pltpu.sync_copy(w0_hbm.at[pl.ds(base, TOK_W)], wb0)
        pltpu.sync_copy(w1_hbm.at[pl.ds(base, TOK_W)], wb1)
        cp0.wait()
        cp1.wait()
        cp0 = pltpu.async_copy(wb0, ws_hbm.at[i0], sem0)
        cp1 = pltpu.async_copy(wb1, ws_hbm.at[i1], sem1)
        cp0.wait()
        cp1.wait()

    return _dispatch


def _ffn_body(cnt_ref, xs_ref, ws_ref, w1_ref, b1_ref, w2_ref, b2_ref,
              ys_ref):
    e = pl.program_id(0)
    f = pl.program_id(1)
    cnt = cnt_ref[e]
    trips = lax.div(cnt + (CHUNK - 1), CHUNK)
    w1b = w1_ref[0].astype(jnp.bfloat16)
    w2b = w2_ref[0].astype(jnp.bfloat16)

    def body(i, carry):
        rs = pl.ds(i * CHUNK, CHUNK)
        xb = xs_ref[rs, :].astype(jnp.bfloat16)
        h = lax.dot_general(xb, w1b, (((1,), (0,)), ((), ())),
                            preferred_element_type=jnp.float32)
        h = h + b1_ref[0]
        h = 0.5 * h * (1.0 + lax.erf(h * 0.7071067811865476))
        part = lax.dot_general(h.astype(jnp.bfloat16), w2b,
                               (((1,), (0,)), ((), ())),
                               preferred_element_type=jnp.float32)

        @pl.when(f == 0)
        def _():
            ys_ref[rs, :] = part + b2_ref[0]

        @pl.when(f > 0)
        def _():
            ys_ref[rs, :] += part

        @pl.when(f == NF - 1)
        def _():
            ys_ref[rs, :] *= ws_ref[rs, 0:1]

        return carry

    lax.fori_loop(0, trips, body, 0)


def _ffn_call(counts, xs, ws, W1, b1r, W2, b2r):
    grid_spec = pltpu.PrefetchScalarGridSpec(
        num_scalar_prefetch=1,
        grid=(E, NF),
        in_specs=[
            pl.BlockSpec((SEG, D), lambda e, f, cnt: (e, 0)),
            pl.BlockSpec((SEG, WREP), lambda e, f, cnt: (e, 0)),
            pl.BlockSpec((1, D, FBLK), lambda e, f, cnt: (e, 0, f)),
            pl.BlockSpec((1, 1, FBLK), lambda e, f, cnt: (e, 0, f)),
            pl.BlockSpec((1, FBLK, D), lambda e, f, cnt: (e, f, 0)),
            pl.BlockSpec((1, 1, D), lambda e, f, cnt: (e, 0, 0)),
        ],
        out_specs=pl.BlockSpec((SEG, D), lambda e, f, cnt: (e, 0)),
    )
    return pl.pallas_call(
        _ffn_body,
        grid_spec=grid_spec,
        out_shape=jax.ShapeDtypeStruct((E * SEG, D), jnp.float32),
        compiler_params=pltpu.CompilerParams(vmem_limit_bytes=64 * 1024 * 1024),
    )(counts, xs, ws, W1, b1r, W2, b2r)


@functools.lru_cache(maxsize=1)
def _make_combine():
    mesh = plsc.VectorSubcoreMesh(core_axis_name="c", subcore_axis_name="s")

    @functools.partial(
        pl.kernel, mesh=mesh,
        out_type=jax.ShapeDtypeStruct((N, D), jnp.float32),
        scratch_types=[pltpu.VMEM((CSUB, D), jnp.float32),
                       pltpu.VMEM((CSUB, D), jnp.float32),
                       pltpu.VMEM((CSUB,), jnp.int32),
                       pltpu.VMEM((CSUB,), jnp.int32),
                       pltpu.SemaphoreType.DMA,
                       pltpu.SemaphoreType.DMA])
    def _combine(ys_hbm, p0_hbm, p1_hbm, out_hbm, b0, b1, i0, i1, sem0, sem1):
        wid = lax.axis_index("s") * NC + lax.axis_index("c")
        base = wid * TOK_W
        for sub in range(TOK_W // CSUB):
            sb = base + sub * CSUB
            pltpu.sync_copy(p0_hbm.at[pl.ds(sb, CSUB)], i0)
            pltpu.sync_copy(p1_hbm.at[pl.ds(sb, CSUB)], i1)
            cp0 = pltpu.async_copy(ys_hbm.at[i0], b0, sem0)
            cp1 = pltpu.async_copy(ys_hbm.at[i1], b1, sem1)
            cp0.wait()
            cp1.wait()

            @pl.loop(0, CSUB)
            def _(r):
                @pl.loop(0, D, step=16)
                def _(cc):
                    b0[r, pl.ds(cc, 16)] = (b0[r, pl.ds(cc, 16)]
                                            + b1[r, pl.ds(cc, 16)])

            pltpu.sync_copy(b0, out_hbm.at[pl.ds(sb, CSUB)])

    return _combine


@jax.jit
def kernel(x, Wg, W1, b1, W2, b2):
    x2 = x.reshape(N, D)
    aux, cnt2, pos0c, pos1c, w0r, w1r = _gate_call(x2, Wg)
    counts = cnt2.reshape(E)
    pos0 = pos0c.reshape(N)
    pos1 = pos1c.reshape(N)
    xs, ws = _make_dispatch()(x2, pos0, pos1, w0r, w1r)
    ys = _ffn_call(counts, xs, ws, W1, b1.reshape(E, 1, DFF), W2,
                   b2.reshape(E, 1, D))
    out = _make_combine()(ys, pos0, pos1)
    return out.reshape(B, L, D), aux[0, 0]
